# Initial kernel scaffold; baseline (speedup 1.0000x reference)
import jax
import jax.numpy as jnp
from jax.experimental import pallas as pl


def kernel(logits, G, W1, b1, W2, b2, W3, b3, W4, b4, W5, b5):
    raise NotImplementedError("write your pallas kernel here")


# speedup vs baseline: 3.3208x; 3.3208x over previous
import functools

import jax
import jax.numpy as jnp
from jax import lax
from jax.experimental import pallas as pl
from jax.experimental.pallas import tpu as pltpu
from jax.experimental.pallas import tpu_sc as plsc

N_NODES = 50000
D = 128
NEIGH = 7
NW = 32
B = 112
NBLK = 14
PER_W = B * NBLK
NPAD = NW * PER_W
MT = 512


def _selu(x):
    scale = 1.0507009873554805
    alpha = 1.6732632423543772
    return scale * jnp.where(x > 0, x, alpha * (jnp.exp(x) - 1.0))


def _mm7_body(x_ref, w_ref, o_ref):
    x = x_ref[...]
    for k in range(NEIGH):
        o_ref[k] = jnp.dot(x, w_ref[k], preferred_element_type=jnp.float32)


def _mm7(x, wstack):
    nb = NPAD // MT
    return pl.pallas_call(
        _mm7_body,
        grid=(nb,),
        in_specs=[
            pl.BlockSpec((MT, D), lambda i: (i, 0)),
            pl.BlockSpec((NEIGH, D, D), lambda i: (0, 0, 0)),
        ],
        out_specs=pl.BlockSpec((NEIGH, MT, D), lambda i: (0, i, 0)),
        out_shape=jax.ShapeDtypeStruct((NEIGH, NPAD, D), jnp.float32),
    )(x, wstack)


def _mid_body(h_ref, b1_ref, w2_ref, b2_ref, w3_ref, o_ref):
    x1 = _selu(h_ref[...] + b1_ref[...])
    t = _selu(jnp.dot(x1, w2_ref[...], preferred_element_type=jnp.float32)
              + b2_ref[...])
    for k in range(NEIGH):
        o_ref[k] = jnp.dot(t, w3_ref[k], preferred_element_type=jnp.float32)


def _mid(h, b1, w2, b2, w3stack):
    nb = NPAD // MT
    return pl.pallas_call(
        _mid_body,
        grid=(nb,),
        in_specs=[
            pl.BlockSpec((MT, D), lambda i: (i, 0)),
            pl.BlockSpec((1, D), lambda i: (0, 0)),
            pl.BlockSpec((D, D), lambda i: (0, 0)),
            pl.BlockSpec((1, D), lambda i: (0, 0)),
            pl.BlockSpec((NEIGH, D, D), lambda i: (0, 0, 0)),
        ],
        out_specs=pl.BlockSpec((NEIGH, MT, D), lambda i: (0, i, 0)),
        out_shape=jax.ShapeDtypeStruct((NEIGH, NPAD, D), jnp.float32),
    )(h, b1, w2, b2, w3stack)


def _out_body(h_ref, b3_ref, w4_ref, b4_ref, w5_ref, b5_ref, o_ref):
    x2 = _selu(h_ref[...] + b3_ref[...])
    t = _selu(jnp.dot(x2, w4_ref[...], preferred_element_type=jnp.float32)
              + b4_ref[...])
    z = jnp.dot(t, w5_ref[...], preferred_element_type=jnp.float32) + b5_ref[...]
    o_ref[...] = 1.0 / (1.0 + jnp.exp(-z))


def _out_stage(h, b3, w4, b4, w5two, b5two):
    nb = NPAD // MT
    return pl.pallas_call(
        _out_body,
        grid=(nb,),
        in_specs=[
            pl.BlockSpec((MT, D), lambda i: (i, 0)),
            pl.BlockSpec((1, D), lambda i: (0, 0)),
            pl.BlockSpec((D, D), lambda i: (0, 0)),
            pl.BlockSpec((1, D), lambda i: (0, 0)),
            pl.BlockSpec((D, 8), lambda i: (0, 0)),
            pl.BlockSpec((1, 8), lambda i: (0, 0)),
        ],
        out_specs=pl.BlockSpec((MT, 8), lambda i: (i, 0)),
        out_shape=jax.ShapeDtypeStruct((NPAD, 8), jnp.float32),
    )(h, b3, w4, b4, w5two, b5two)


def _sc_mesh():
    return plsc.VectorSubcoreMesh(core_axis_name="c", subcore_axis_name="s")


def _accum(acc, g):
    @pl.loop(0, B)
    def _row(r):
        for j in range(D // 16):
            sl = (pl.ds(r, 1), pl.ds(j * 16, 16))
            acc[sl] = acc[sl] + g[sl]


def _gather_sum(table, idx_blocks):
    @functools.partial(
        pl.kernel,
        out_type=jax.ShapeDtypeStruct((NPAD, D), jnp.float32),
        mesh=_sc_mesh(),
        scratch_types=[
            pltpu.VMEM((NEIGH, B), jnp.int32),
            pltpu.VMEM((B, D), jnp.float32),
            pltpu.VMEM((B, D), jnp.float32),
            pltpu.VMEM((B, D), jnp.float32),
            pltpu.VMEM((B, D), jnp.float32),
            pltpu.VMEM((B, D), jnp.float32),
            pltpu.SemaphoreType.DMA,
            pltpu.SemaphoreType.DMA,
            pltpu.SemaphoreType.DMA,
            pltpu.SemaphoreType.DMA,
            pltpu.SemaphoreType.DMA,
        ],
    )
    def k(table_hbm, idx_hbm, out_hbm, idx_v, acc, g0, g1, g2, g3,
          sa, s0, s1, s2, s3):
        cid = lax.axis_index("c")
        sid = lax.axis_index("s")
        wid = sid * 2 + cid

        @pl.loop(0, NBLK)
        def _blk(blk):
            base = wid * PER_W + blk * B
            pltpu.sync_copy(idx_hbm.at[wid * NBLK + blk], idx_v)
            ca = pltpu.async_copy(table_hbm.at[idx_v.at[0]], acc, sa)
            c0 = pltpu.async_copy(table_hbm.at[idx_v.at[1]], g0, s0)
            c1 = pltpu.async_copy(table_hbm.at[idx_v.at[2]], g1, s1)
            c2 = pltpu.async_copy(table_hbm.at[idx_v.at[3]], g2, s2)
            c3 = pltpu.async_copy(table_hbm.at[idx_v.at[4]], g3, s3)
            ca.wait()
            c0.wait()
            _accum(acc, g0)
            c5 = pltpu.async_copy(table_hbm.at[idx_v.at[5]], g0, s0)
            c1.wait()
            _accum(acc, g1)
            c6 = pltpu.async_copy(table_hbm.at[idx_v.at[6]], g1, s1)
            c2.wait()
            _accum(acc, g2)
            c3.wait()
            _accum(acc, g3)
            c5.wait()
            _accum(acc, g0)
            c6.wait()
            _accum(acc, g1)
            pltpu.sync_copy(acc, out_hbm.at[pl.ds(base, B)])

    return k(table, idx_blocks)


def kernel(logits, G, W1, b1, W2, b2, W3, b3, W4, b4, W5, b5):
    f32 = jnp.float32
    x = logits.astype(f32)
    npadrows = NPAD - N_NODES
    x = jnp.concatenate([x, jnp.zeros((npadrows, D), f32)], axis=0)

    Gi = G.astype(jnp.int32)
    gpad = (jnp.arange(npadrows, dtype=jnp.int32)[:, None] * NEIGH
            + jnp.arange(NEIGH, dtype=jnp.int32)[None, :]) % N_NODES
    Gp = jnp.concatenate([Gi, gpad], axis=0)
    idxT = (Gp.T + (jnp.arange(NEIGH, dtype=jnp.int32) * NPAD)[:, None])
    idxT = idxT.astype(jnp.int32)
    idx_blocks = (idxT.reshape(NEIGH, NW, NBLK, B)
                  .transpose(1, 2, 0, 3)
                  .reshape(NW * NBLK, NEIGH, B))

    w1s = W1.astype(f32).reshape(NEIGH, D, D)
    w3s = W3.astype(f32).reshape(NEIGH, D, D)
    b1r = b1.astype(f32).reshape(1, D)
    b2r = b2.astype(f32).reshape(1, D)
    b3r = b3.astype(f32).reshape(1, D)
    b4r = b4.astype(f32).reshape(1, D)
    wd = jnp.stack([W5[:, 0] - W5[:, 1], W5[:, 1] - W5[:, 0]], axis=1)
    w5two = jnp.concatenate([wd.astype(f32), jnp.zeros((D, 6), f32)], axis=1)
    bd = jnp.stack([b5[0] - b5[1], b5[1] - b5[0]])
    b5two = jnp.concatenate([bd.astype(f32), jnp.zeros((6,), f32)])
    b5two = b5two.reshape(1, 8)

    y1 = _mm7(x, w1s).reshape(NEIGH * NPAD, D)
    h1 = _gather_sum(y1, idx_blocks)
    y2 = _mid(h1, b1r, W2.astype(f32), b2r, w3s).reshape(NEIGH * NPAD, D)
    h2 = _gather_sum(y2, idx_blocks)
    out = _out_stage(h2, b3r, W4.astype(f32), b4r, w5two, b5two)
    return out[: N_NODES - 1, :2]
